# baseline (speedup 1.0000x reference)
import functools

import jax
import jax.numpy as jnp
from jax import lax
from jax.experimental import pallas as pl
from jax.experimental.pallas import tpu as pltpu
from jax.experimental.pallas import tpu_sc as plsc

_B, _C, _N = 4, 768, 8192
_K = 8
_ROWS = _B * _C
_NW = 32
_RS = 1792
_RT = _ROWS - _RS
_RPW = _RS // _NW
_LANES = 16
_VPR = _N // _LANES
_OCTS = _VPR // 8
_FB_UNROLL = 4


def _vsort_desc(v):
    return plsc.sort_key_val(v, v, descending=True)[0]


def _combine(a, b, lane_lt8):
    return _vsort_desc(jnp.where(lane_lt8, a, lax.rev(b, (0,))))


def _insert(ts, x):
    out = []
    cur = x
    for t in ts:
        out.append(jnp.maximum(t, cur))
        cur = jnp.minimum(t, cur)
    return tuple(out)


def _merge_tree(vs, lane_lt8):
    vs = [_vsort_desc(t) for t in vs]
    while len(vs) > 1:
        vs = [_combine(vs[i], vs[i + 1], lane_lt8)
              for i in range(0, len(vs), 2)]
    return vs[0]


def _quad(a, b, c, d):
    m1, n1 = jnp.maximum(a, b), jnp.minimum(a, b)
    m2, n2 = jnp.maximum(c, d), jnp.minimum(c, d)
    w = jnp.maximum(m1, m2)
    sec = jnp.maximum(jnp.minimum(m1, m2), jnp.maximum(n1, n2))
    return w, sec


@functools.partial(
    pl.kernel,
    out_type=jax.ShapeDtypeStruct((_RS * _K,), jnp.float32),
    mesh=plsc.VectorSubcoreMesh(core_axis_name="c", subcore_axis_name="s"),
    scratch_types=[
        pltpu.VMEM((_N,), jnp.float32),
        pltpu.VMEM((_N,), jnp.float32),
        pltpu.VMEM((_LANES,), jnp.float32),
        pltpu.VMEM((_RPW * _K + _LANES - _K,), jnp.float32),
        pltpu.SemaphoreType.DMA,
        pltpu.SemaphoreType.DMA,
    ],
    compiler_params=pltpu.CompilerParams(needs_layout_passes=False),
)
def _topk_sc(x_hbm, out_hbm, buf0, buf1, s16, out_v, sem0, sem1):
    nc = 2
    wid = lax.axis_index("s") * nc + lax.axis_index("c")
    base = wid * _RPW
    lane = lax.iota(jnp.int32, 16)
    lane_lt8 = lane < _K
    seven = jnp.full((_LANES,), 7, jnp.int32)
    neg = jnp.full((_LANES,), -jnp.inf, jnp.float32)

    def row_compute(buf, row_local):
        def body(i, carry):
            ts, dmax = carry[:4], carry[4]
            off = i * 8 * _LANES
            v = [buf[pl.ds(off + j * _LANES, _LANES)] for j in range(8)]
            w1, s1 = _quad(*v[:4])
            w2, s2 = _quad(*v[4:])
            w = jnp.maximum(w1, w2)
            sec = jnp.maximum(jnp.minimum(w1, w2), jnp.maximum(s1, s2))
            dmax = jnp.maximum(dmax, sec)
            ts = _insert(ts, w)
            return ts + (dmax,)

        carry = lax.fori_loop(0, _OCTS, body, (neg,) * 5)
        ts, dmax = carry[:4], carry[4]
        cand = _merge_tree(list(ts), lane_lt8)
        s16[...] = cand
        out8 = plsc.load_gather(s16, [seven])
        viol = jnp.any((ts[3] > out8) | (dmax > out8))

        def fallback():
            def fb_body(i, ts8):
                for j in range(_FB_UNROLL):
                    v = buf[pl.ds((i * _FB_UNROLL + j) * _LANES, _LANES)]
                    ts8 = _insert(ts8, v)
                return ts8
            ts8 = lax.fori_loop(0, _VPR // _FB_UNROLL, fb_body, (neg,) * _K)
            return _merge_tree(list(ts8), lane_lt8)

        final = lax.cond(viol, fallback, lambda: cand)
        plsc.store_compressed(out_v.at[pl.ds(row_local * _K, _LANES)],
                              final, mask=lane_lt8)

    pltpu.async_copy(x_hbm.at[base], buf0, sem0)
    pltpu.async_copy(x_hbm.at[base + 1], buf1, sem1)

    def step(st, carry):
        r0 = 2 * st
        pltpu.make_async_copy(x_hbm.at[base + r0], buf0, sem0).wait()
        row_compute(buf0, r0)
        nxt0 = jnp.minimum(r0 + 2, _RPW - 1)
        pltpu.async_copy(x_hbm.at[base + nxt0], buf0, sem0)

        pltpu.make_async_copy(x_hbm.at[base + r0 + 1], buf1, sem1).wait()
        row_compute(buf1, r0 + 1)
        nxt1 = jnp.minimum(r0 + 3, _RPW - 1)
        pltpu.async_copy(x_hbm.at[base + nxt1], buf1, sem1)
        return carry

    lax.fori_loop(0, _RPW // 2, step, 0)

    pltpu.make_async_copy(x_hbm.at[base], buf0, sem0).wait()
    pltpu.make_async_copy(x_hbm.at[base], buf1, sem1).wait()

    pltpu.sync_copy(out_v.at[pl.ds(0, _RPW * _K)],
                    out_hbm.at[pl.ds(base * _K, _RPW * _K)])


def _tc_body(x_ref, o_ref):
    neg = jnp.full((8, 128), -jnp.inf, jnp.float32)
    ts = (neg,) * _K
    for j in range(_N // 128):
        ts = _insert(ts, x_ref[:, j * 128:(j + 1) * 128])
    ts = list(ts)
    for shift in (64, 32, 16, 8, 4, 2, 1):
        us = [pltpu.roll(t, shift, axis=1) for t in ts]
        m = [jnp.maximum(ts[i], us[_K - 1 - i]) for i in range(_K)]
        for dist in (4, 2, 1):
            nl = list(m)
            for bs in range(0, _K, 2 * dist):
                for i in range(bs, bs + dist):
                    nl[i] = jnp.maximum(m[i], m[i + dist])
                    nl[i + dist] = jnp.minimum(m[i], m[i + dist])
            m = nl
        ts = m
    lanes = lax.broadcasted_iota(jnp.int32, (8, 128), 1)
    res = ts[0]
    for k in range(1, _K):
        res = jnp.where(lanes == k, ts[k], res)
    o_ref[...] = res


_topk_tc = pl.pallas_call(
    _tc_body,
    grid=(_RT // 8,),
    in_specs=[pl.BlockSpec((8, _N), lambda i: (_RS // 8 + i, 0))],
    out_specs=pl.BlockSpec((8, 128), lambda i: (i, 0)),
    out_shape=jax.ShapeDtypeStruct((_RT, 128), jnp.float32),
)


def kernel(x):
    xf = x.reshape(_ROWS, _N)
    y_sc = _topk_sc(xf).reshape(_RS, _K)
    y_tc = _topk_tc(xf)[:, :_K]
    return jnp.concatenate([y_sc, y_tc], axis=0).reshape(_B, _C, _K)


# speedup vs baseline: 1.5761x; 1.5761x over previous
import functools

import jax
import jax.numpy as jnp
from jax import lax
from jax.experimental import pallas as pl
from jax.experimental.pallas import tpu as pltpu
from jax.experimental.pallas import tpu_sc as plsc

_B, _C, _N = 4, 768, 8192
_K = 8
_ROWS = _B * _C
_NW = 32
_RPW = _ROWS // _NW
_LANES = 16
_VPR = _N // _LANES
_OCTS = _VPR // 8
_FB_UNROLL = 4


def _vsort_desc(v):
    return plsc.sort_key_val(v, v, descending=True)[0]


def _combine(a, b, lane_lt8):
    return _vsort_desc(jnp.where(lane_lt8, a, lax.rev(b, (0,))))


def _insert(ts, x):
    out = []
    cur = x
    for t in ts:
        out.append(jnp.maximum(t, cur))
        cur = jnp.minimum(t, cur)
    return tuple(out)


def _merge_tree(vs, lane_lt8):
    vs = [_vsort_desc(t) for t in vs]
    while len(vs) > 1:
        vs = [_combine(vs[i], vs[i + 1], lane_lt8)
              for i in range(0, len(vs), 2)]
    return vs[0]


def _quad(a, b, c, d):
    m1, n1 = jnp.maximum(a, b), jnp.minimum(a, b)
    m2, n2 = jnp.maximum(c, d), jnp.minimum(c, d)
    w = jnp.maximum(m1, m2)
    sec = jnp.maximum(jnp.minimum(m1, m2), jnp.maximum(n1, n2))
    return w, sec


@functools.partial(
    pl.kernel,
    out_type=jax.ShapeDtypeStruct((_ROWS * _K,), jnp.float32),
    mesh=plsc.VectorSubcoreMesh(core_axis_name="c", subcore_axis_name="s"),
    scratch_types=[
        pltpu.VMEM((_N,), jnp.float32),
        pltpu.VMEM((_N,), jnp.float32),
        pltpu.VMEM((_LANES,), jnp.float32),
        pltpu.VMEM((_RPW * _K + _LANES - _K,), jnp.float32),
        pltpu.SemaphoreType.DMA,
        pltpu.SemaphoreType.DMA,
    ],
    compiler_params=pltpu.CompilerParams(needs_layout_passes=False),
)
def _topk_sc(x_hbm, out_hbm, buf0, buf1, s16, out_v, sem0, sem1):
    nc = 2
    wid = lax.axis_index("s") * nc + lax.axis_index("c")
    base = wid * _RPW
    lane = lax.iota(jnp.int32, 16)
    lane_lt8 = lane < _K
    seven = jnp.full((_LANES,), 7, jnp.int32)
    neg = jnp.full((_LANES,), -jnp.inf, jnp.float32)

    def row_compute(buf, row_local):
        def body(i, carry):
            ts, dmax = carry[:4], carry[4]
            for u in range(2):
                off = (2 * i + u) * 8 * _LANES
                v = [buf[pl.ds(off + j * _LANES, _LANES)] for j in range(8)]
                w1, s1 = _quad(*v[:4])
                w2, s2 = _quad(*v[4:])
                w = jnp.maximum(w1, w2)
                sec = jnp.maximum(jnp.minimum(w1, w2), jnp.maximum(s1, s2))
                dmax = jnp.maximum(dmax, sec)
                ts = _insert(ts, w)
            return ts + (dmax,)

        carry = lax.fori_loop(0, _OCTS // 2, body, (neg,) * 5)
        ts, dmax = carry[:4], carry[4]
        cand = _merge_tree(list(ts), lane_lt8)
        s16[...] = cand
        out8 = plsc.load_gather(s16, [seven])
        viol = jnp.any((ts[3] > out8) | (dmax > out8))

        def fallback():
            def fb_body(i, ts8):
                for j in range(_FB_UNROLL):
                    v = buf[pl.ds((i * _FB_UNROLL + j) * _LANES, _LANES)]
                    ts8 = _insert(ts8, v)
                return ts8
            ts8 = lax.fori_loop(0, _VPR // _FB_UNROLL, fb_body, (neg,) * _K)
            return _merge_tree(list(ts8), lane_lt8)

        final = lax.cond(viol, fallback, lambda: cand)
        plsc.store_compressed(out_v.at[pl.ds(row_local * _K, _LANES)],
                              final, mask=lane_lt8)

    pltpu.async_copy(x_hbm.at[base], buf0, sem0)
    pltpu.async_copy(x_hbm.at[base + 1], buf1, sem1)

    def step(st, carry):
        r0 = 2 * st
        pltpu.make_async_copy(x_hbm.at[base + r0], buf0, sem0).wait()
        row_compute(buf0, r0)
        nxt0 = jnp.minimum(r0 + 2, _RPW - 1)
        pltpu.async_copy(x_hbm.at[base + nxt0], buf0, sem0)

        pltpu.make_async_copy(x_hbm.at[base + r0 + 1], buf1, sem1).wait()
        row_compute(buf1, r0 + 1)
        nxt1 = jnp.minimum(r0 + 3, _RPW - 1)
        pltpu.async_copy(x_hbm.at[base + nxt1], buf1, sem1)
        return carry

    lax.fori_loop(0, _RPW // 2, step, 0)

    pltpu.make_async_copy(x_hbm.at[base], buf0, sem0).wait()
    pltpu.make_async_copy(x_hbm.at[base], buf1, sem1).wait()

    pltpu.sync_copy(out_v.at[pl.ds(0, _RPW * _K)],
                    out_hbm.at[pl.ds(base * _K, _RPW * _K)])


def kernel(x):
    out = _topk_sc(x.reshape(_ROWS, _N))
    return out.reshape(_B, _C, _K)


# speedup vs baseline: 1.9318x; 1.2257x over previous
import functools

import jax
import jax.numpy as jnp
from jax import lax
from jax.experimental import pallas as pl
from jax.experimental.pallas import tpu as pltpu
from jax.experimental.pallas import tpu_sc as plsc

_B, _C, _N = 4, 768, 8192
_K = 8
_ROWS = _B * _C
_NW = 32
_RPW = _ROWS // _NW
_LANES = 16
_VPR = _N // _LANES
_OCTS = _VPR // 8
_FB_UNROLL = 4


def _vsort_desc(v):
    return plsc.sort_key_val(v, v, descending=True)[0]


def _combine(a, b, lane_lt8):
    return _vsort_desc(jnp.where(lane_lt8, a, lax.rev(b, (0,))))


def _insert(ts, x):
    out = []
    cur = x
    for t in ts:
        out.append(jnp.maximum(t, cur))
        cur = jnp.minimum(t, cur)
    return tuple(out)


def _merge_tree(vs, lane_lt8):
    vs = [_vsort_desc(t) for t in vs]
    while len(vs) > 1:
        vs = [_combine(vs[i], vs[i + 1], lane_lt8)
              for i in range(0, len(vs), 2)]
    return vs[0]


def _quad(a, b, c, d):
    m1, n1 = jnp.maximum(a, b), jnp.minimum(a, b)
    m2, n2 = jnp.maximum(c, d), jnp.minimum(c, d)
    w = jnp.maximum(m1, m2)
    sec = jnp.maximum(jnp.minimum(m1, m2), jnp.maximum(n1, n2))
    return w, sec


@functools.partial(
    pl.kernel,
    out_type=jax.ShapeDtypeStruct((_ROWS * _K,), jnp.float32),
    mesh=plsc.VectorSubcoreMesh(core_axis_name="c", subcore_axis_name="s"),
    scratch_types=[
        pltpu.VMEM((_N,), jnp.float32),
        pltpu.VMEM((_N,), jnp.float32),
        pltpu.VMEM((_N,), jnp.float32),
        pltpu.VMEM((_LANES,), jnp.float32),
        pltpu.VMEM((_RPW * _K + _LANES - _K,), jnp.float32),
        pltpu.SemaphoreType.DMA,
        pltpu.SemaphoreType.DMA,
        pltpu.SemaphoreType.DMA,
    ],
    compiler_params=pltpu.CompilerParams(needs_layout_passes=False),
)
def _topk_sc(x_hbm, out_hbm, buf0, buf1, buf2, s16, out_v, sem0, sem1, sem2):
    nc = 2
    wid = lax.axis_index("s") * nc + lax.axis_index("c")
    base = wid * _RPW
    lane = lax.iota(jnp.int32, 16)
    lane_lt8 = lane < _K
    seven = jnp.full((_LANES,), 7, jnp.int32)
    neg = jnp.full((_LANES,), -jnp.inf, jnp.float32)

    def row_compute(buf, row_local):
        def body(i, carry):
            ts, dmax = carry[:4], carry[4]
            for u in range(2):
                off = (2 * i + u) * 8 * _LANES
                v = [buf[pl.ds(off + j * _LANES, _LANES)] for j in range(8)]
                w1, s1 = _quad(*v[:4])
                w2, s2 = _quad(*v[4:])
                w = jnp.maximum(w1, w2)
                sec = jnp.maximum(jnp.minimum(w1, w2), jnp.maximum(s1, s2))
                dmax = jnp.maximum(dmax, sec)
                ts = _insert(ts, w)
            return ts + (dmax,)

        carry = lax.fori_loop(0, _OCTS // 2, body, (neg,) * 5)
        ts, dmax = carry[:4], carry[4]
        cand = _merge_tree(list(ts), lane_lt8)
        s16[...] = cand
        out8 = plsc.load_gather(s16, [seven])
        viol = jnp.any((ts[3] > out8) | (dmax > out8))

        def fallback():
            def fb_body(i, ts8):
                for j in range(_FB_UNROLL):
                    v = buf[pl.ds((i * _FB_UNROLL + j) * _LANES, _LANES)]
                    ts8 = _insert(ts8, v)
                return ts8
            ts8 = lax.fori_loop(0, _VPR // _FB_UNROLL, fb_body, (neg,) * _K)
            return _merge_tree(list(ts8), lane_lt8)

        final = lax.cond(viol, fallback, lambda: cand)
        plsc.store_compressed(out_v.at[pl.ds(row_local * _K, _LANES)],
                              final, mask=lane_lt8)

    bufs = (buf0, buf1, buf2)
    sems = (sem0, sem1, sem2)
    for b in range(3):
        pltpu.async_copy(x_hbm.at[base + b], bufs[b], sems[b])

    def step(st, carry):
        r0 = 3 * st
        for b in range(3):
            pltpu.make_async_copy(x_hbm.at[base + r0 + b], bufs[b],
                                  sems[b]).wait()
            row_compute(bufs[b], r0 + b)
            nxt = jnp.minimum(r0 + b + 3, _RPW - 1)
            pltpu.async_copy(x_hbm.at[base + nxt], bufs[b], sems[b])
        return carry

    lax.fori_loop(0, _RPW // 3, step, 0)

    for b in range(3):
        pltpu.make_async_copy(x_hbm.at[base], bufs[b], sems[b]).wait()

    pltpu.sync_copy(out_v.at[pl.ds(0, _RPW * _K)],
                    out_hbm.at[pl.ds(base * _K, _RPW * _K)])


def kernel(x):
    out = _topk_sc(x.reshape(_ROWS, _N))
    return out.reshape(_B, _C, _K)


# speedup vs baseline: 2.0443x; 1.0583x over previous
import functools

import jax
import jax.numpy as jnp
from jax import lax
from jax.experimental import pallas as pl
from jax.experimental.pallas import tpu as pltpu
from jax.experimental.pallas import tpu_sc as plsc

_B, _C, _N = 4, 768, 8192
_K = 8
_ROWS = _B * _C
_NW = 32
_RPW = _ROWS // _NW
_LANES = 16
_VPR = _N // _LANES
_OCTS = _VPR // 8
_FB_UNROLL = 4


def _vsort_desc(v):
    return plsc.sort_key_val(v, v, descending=True)[0]


def _combine(a, b, lane_lt8):
    return _vsort_desc(jnp.where(lane_lt8, a, lax.rev(b, (0,))))


def _insert(ts, x):
    out = []
    cur = x
    for t in ts:
        out.append(jnp.maximum(t, cur))
        cur = jnp.minimum(t, cur)
    return tuple(out)


def _merge_tree(vs, lane_lt8):
    vs = [_vsort_desc(t) for t in vs]
    while len(vs) > 1:
        vs = [_combine(vs[i], vs[i + 1], lane_lt8)
              for i in range(0, len(vs), 2)]
    return vs[0]


def _quad(a, b, c, d):
    m1, n1 = jnp.maximum(a, b), jnp.minimum(a, b)
    m2, n2 = jnp.maximum(c, d), jnp.minimum(c, d)
    w = jnp.maximum(m1, m2)
    sec = jnp.maximum(jnp.minimum(m1, m2), jnp.maximum(n1, n2))
    return w, sec


@functools.partial(
    pl.kernel,
    out_type=jax.ShapeDtypeStruct((_ROWS * _K,), jnp.float32),
    mesh=plsc.VectorSubcoreMesh(core_axis_name="c", subcore_axis_name="s"),
    scratch_types=[
        pltpu.VMEM((_N,), jnp.float32),
        pltpu.VMEM((_N,), jnp.float32),
        pltpu.VMEM((_N,), jnp.float32),
        pltpu.VMEM((_N,), jnp.float32),
        pltpu.VMEM((_LANES,), jnp.float32),
        pltpu.VMEM((_RPW * _K + _LANES - _K,), jnp.float32),
        pltpu.SemaphoreType.DMA,
        pltpu.SemaphoreType.DMA,
        pltpu.SemaphoreType.DMA,
        pltpu.SemaphoreType.DMA,
    ],
    compiler_params=pltpu.CompilerParams(needs_layout_passes=False),
)
def _topk_sc(x_hbm, out_hbm, buf0, buf1, buf2, buf3, s16, out_v, sem0, sem1, sem2, sem3):
    nc = 2
    wid = lax.axis_index("s") * nc + lax.axis_index("c")
    base = wid * _RPW
    lane = lax.iota(jnp.int32, 16)
    lane_lt8 = lane < _K
    seven = jnp.full((_LANES,), 7, jnp.int32)
    neg = jnp.full((_LANES,), -jnp.inf, jnp.float32)

    def row_compute(buf, row_local):
        def body(i, carry):
            ts, dmax = carry[:4], carry[4]
            for u in range(2):
                off = (2 * i + u) * 8 * _LANES
                v = [buf[pl.ds(off + j * _LANES, _LANES)] for j in range(8)]
                w1, s1 = _quad(*v[:4])
                w2, s2 = _quad(*v[4:])
                w = jnp.maximum(w1, w2)
                sec = jnp.maximum(jnp.minimum(w1, w2), jnp.maximum(s1, s2))
                dmax = jnp.maximum(dmax, sec)
                ts = _insert(ts, w)
            return ts + (dmax,)

        carry = lax.fori_loop(0, _OCTS // 2, body, (neg,) * 5)
        ts, dmax = carry[:4], carry[4]
        cand = _merge_tree(list(ts), lane_lt8)
        s16[...] = cand
        out8 = plsc.load_gather(s16, [seven])
        viol = jnp.any((ts[3] > out8) | (dmax > out8))

        def fallback():
            def fb_body(i, ts8):
                for j in range(_FB_UNROLL):
                    v = buf[pl.ds((i * _FB_UNROLL + j) * _LANES, _LANES)]
                    ts8 = _insert(ts8, v)
                return ts8
            ts8 = lax.fori_loop(0, _VPR // _FB_UNROLL, fb_body, (neg,) * _K)
            return _merge_tree(list(ts8), lane_lt8)

        final = lax.cond(viol, fallback, lambda: cand)
        plsc.store_compressed(out_v.at[pl.ds(row_local * _K, _LANES)],
                              final, mask=lane_lt8)

    bufs = (buf0, buf1, buf2, buf3)
    sems = (sem0, sem1, sem2, sem3)
    for b in range(4):
        pltpu.async_copy(x_hbm.at[base + b], bufs[b], sems[b])

    def step(st, carry):
        r0 = 4 * st
        for b in range(4):
            pltpu.make_async_copy(x_hbm.at[base + r0 + b], bufs[b],
                                  sems[b]).wait()
            row_compute(bufs[b], r0 + b)
            nxt = jnp.minimum(r0 + b + 4, _RPW - 1)
            pltpu.async_copy(x_hbm.at[base + nxt], bufs[b], sems[b])
        return carry

    lax.fori_loop(0, _RPW // 4, step, 0)

    for b in range(4):
        pltpu.make_async_copy(x_hbm.at[base], bufs[b], sems[b]).wait()

    pltpu.sync_copy(out_v.at[pl.ds(0, _RPW * _K)],
                    out_hbm.at[pl.ds(base * _K, _RPW * _K)])


def kernel(x):
    out = _topk_sc(x.reshape(_ROWS, _N))
    return out.reshape(_B, _C, _K)
